# baseline (speedup 1.0000x reference)
import functools

import jax
import jax.numpy as jnp
from jax import lax
from jax.experimental import pallas as pl
from jax.experimental.pallas import tpu as pltpu
from jax.experimental.pallas import tpu_sc as plsc

P = 100000
H = 128
B = 1024

_NC = 2
_NS = 16
_NW = _NC * _NS
_NIDX = 2 * B
_BPW = _NIDX // _NW

_PB = 3072
_NFULL = P // _PB
_RBUF = 4
_WBUF = 2
_MAIN = 28
_NEPI = _NFULL - _MAIN
_NOUTER = _MAIN // _RBUF
_TAILW = P - _NFULL * _PB
_TAILOFF = _NFULL * _PB


def _sc_gather(table, idx_flat):
    mesh = plsc.VectorSubcoreMesh(core_axis_name="c", subcore_axis_name="s")

    @functools.partial(
        pl.kernel,
        mesh=mesh,
        out_type=jax.ShapeDtypeStruct((_NIDX, H), jnp.float32),
        scratch_types=[
            pltpu.VMEM((_BPW,), jnp.int32),
            pltpu.VMEM((_BPW, H), jnp.float32),
            pltpu.SemaphoreType.DMA,
        ],
    )
    def k(table_hbm, idx_hbm, out_hbm, idx_v, rows_v, sem):
        wid = lax.axis_index("s") * _NC + lax.axis_index("c")
        base = wid * _BPW
        pltpu.sync_copy(idx_hbm.at[pl.ds(base, _BPW)], idx_v)
        pltpu.async_copy(table_hbm.at[idx_v], rows_v, sem).wait()
        pltpu.sync_copy(rows_v, out_hbm.at[pl.ds(base, _BPW)])

    return k(table, idx_flat)


def _mlp_body(emb_ref, w1_ref, b1_ref, w2_ref, b2_ref, w3t_hbm, b3_hbm,
              out_hbm, h2t_ref, w3t_bufs, b3_bufs, out_bufs,
              w3t_tail, b3_tail, out_tail,
              rsem, bsem, wsem, tsem):
    def start_read(slot, s):
        pltpu.make_async_copy(
            w3t_hbm.at[pl.ds(s, _PB), :], w3t_bufs.at[slot], rsem.at[slot]
        ).start()
        pltpu.make_async_copy(
            b3_hbm.at[:, pl.ds(s, _PB)], b3_bufs.at[slot], bsem.at[slot]
        ).start()

    for k in range(_RBUF):
        start_read(k, k * _PB)
    pltpu.make_async_copy(
        w3t_hbm.at[pl.ds(_TAILOFF, _TAILW), :], w3t_tail, tsem.at[0]).start()
    pltpu.make_async_copy(
        b3_hbm.at[:, pl.ds(_TAILOFF, _TAILW)], b3_tail, tsem.at[1]).start()

    h1t = jnp.maximum(
        lax.dot_general(w1_ref[...], emb_ref[...], (((0,), (1,)), ((), ())),
                        preferred_element_type=jnp.float32) + b1_ref[...],
        0.0)
    h2t = jnp.maximum(
        lax.dot_general(w2_ref[...], h1t, (((0,), (0,)), ((), ())),
                        preferred_element_type=jnp.float32) + b2_ref[...],
        0.0)
    h2t_ref[...] = h2t.astype(jnp.bfloat16)

    pltpu.make_async_copy(
        w3t_hbm.at[pl.ds(_TAILOFF, _TAILW), :], w3t_tail, tsem.at[0]).wait()
    pltpu.make_async_copy(
        b3_hbm.at[:, pl.ds(_TAILOFF, _TAILW)], b3_tail, tsem.at[1]).wait()
    out_tail[...] = (
        jnp.dot(w3t_tail[...].astype(jnp.bfloat16), h2t_ref[...],
                preferred_element_type=jnp.float32)
        + jnp.transpose(b3_tail[...], (1, 0)))
    pltpu.make_async_copy(
        out_tail, out_hbm.at[pl.ds(_TAILOFF, _TAILW), :], tsem.at[2]).start()

    def wait_read(slot):
        pltpu.make_async_copy(
            w3t_hbm.at[pl.ds(0, _PB), :], w3t_bufs.at[slot], rsem.at[slot]
        ).wait()
        pltpu.make_async_copy(
            b3_hbm.at[:, pl.ds(0, _PB)], b3_bufs.at[slot], bsem.at[slot]
        ).wait()

    def wait_write(slot):
        pltpu.make_async_copy(
            out_bufs.at[slot], out_hbm.at[pl.ds(0, _PB), :], wsem.at[slot]
        ).wait()

    def compute_and_write(rslot, wslot, s):
        out_bufs[wslot] = (
            jnp.dot(w3t_bufs[rslot].astype(jnp.bfloat16), h2t_ref[...],
                    preferred_element_type=jnp.float32)
            + jnp.transpose(b3_bufs[rslot], (1, 0)))
        pltpu.make_async_copy(
            out_bufs.at[wslot], out_hbm.at[pl.ds(s, _PB), :], wsem.at[wslot]
        ).start()

    @pl.loop(0, _NOUTER)
    def _(o):
        base = o * _RBUF
        for k in range(_RBUF):
            i = base + k
            wait_read(k)
            if k >= _WBUF:
                wait_write(k % _WBUF)
            else:
                @pl.when(o > 0)
                def _():
                    wait_write(k % _WBUF)
            compute_and_write(k, k % _WBUF, i * _PB)
            nxt = i + _RBUF
            @pl.when(nxt < _NFULL)
            def _():
                start_read(k, nxt * _PB)

    for e in range(_NEPI):
        wait_read(e)
        wait_write(e % _WBUF)
        compute_and_write(e, e % _WBUF, (_MAIN + e) * _PB)

    for k in range(_WBUF):
        wait_write(k)
    pltpu.make_async_copy(
        out_tail, out_hbm.at[pl.ds(_TAILOFF, _TAILW), :], tsem.at[2]).wait()


def _mlp_tc(emb, W1, b1, W2, b2, W3T, b3):
    out_t = pl.pallas_call(
        _mlp_body,
        in_specs=[
            pl.BlockSpec((B, 2 * H), lambda: (0, 0)),
            pl.BlockSpec((2 * H, H), lambda: (0, 0)),
            pl.BlockSpec((H, 1), lambda: (0, 0)),
            pl.BlockSpec((H, H), lambda: (0, 0)),
            pl.BlockSpec((H, 1), lambda: (0, 0)),
            pl.BlockSpec(memory_space=pl.ANY),
            pl.BlockSpec(memory_space=pl.ANY),
        ],
        out_specs=pl.BlockSpec(memory_space=pl.ANY),
        out_shape=jax.ShapeDtypeStruct((P, B), jnp.float32),
        scratch_shapes=[
            pltpu.VMEM((H, B), jnp.bfloat16),
            pltpu.VMEM((_RBUF, _PB, H), jnp.float32),
            pltpu.VMEM((_RBUF, 1, _PB), jnp.float32),
            pltpu.VMEM((_WBUF, _PB, B), jnp.float32),
            pltpu.VMEM((_TAILW, H), jnp.float32),
            pltpu.VMEM((1, _TAILW), jnp.float32),
            pltpu.VMEM((_TAILW, B), jnp.float32),
            pltpu.SemaphoreType.DMA((_RBUF,)),
            pltpu.SemaphoreType.DMA((_RBUF,)),
            pltpu.SemaphoreType.DMA((_WBUF,)),
            pltpu.SemaphoreType.DMA((3,)),
        ],
    )(emb, W1, b1.reshape(H, 1), W2, b2.reshape(H, 1), W3T,
      b3.reshape(1, P))
    return out_t.T


def kernel(x, table, W1, b1, W2, b2, W3, b3):
    idx_flat = x.reshape(-1).astype(jnp.int32)
    emb = _sc_gather(table, idx_flat).reshape(B, 2 * H)
    return _mlp_tc(emb, W1, b1, W2, b2, W3.T, b3)


# speedup vs baseline: 1.0045x; 1.0045x over previous
import functools

import jax
import jax.numpy as jnp
from jax import lax
from jax.experimental import pallas as pl
from jax.experimental.pallas import tpu as pltpu
from jax.experimental.pallas import tpu_sc as plsc

P = 100000
H = 128
B = 1024

_NC = 2
_NS = 16
_NW = _NC * _NS
_NIDX = 2 * B
_BPW = _NIDX // _NW

_PB = 3072
_NFULL = P // _PB
_RBUF = 3
_WBUF = 3
_MAIN = 30
_NEPI = _NFULL - _MAIN
_NOUTER = _MAIN // _RBUF
_TAILW = P - _NFULL * _PB
_TAILOFF = _NFULL * _PB


def _sc_gather(table, idx_flat):
    mesh = plsc.VectorSubcoreMesh(core_axis_name="c", subcore_axis_name="s")

    @functools.partial(
        pl.kernel,
        mesh=mesh,
        out_type=jax.ShapeDtypeStruct((_NIDX, H), jnp.float32),
        scratch_types=[
            pltpu.VMEM((_BPW,), jnp.int32),
            pltpu.VMEM((_BPW, H), jnp.float32),
            pltpu.SemaphoreType.DMA,
        ],
    )
    def k(table_hbm, idx_hbm, out_hbm, idx_v, rows_v, sem):
        wid = lax.axis_index("s") * _NC + lax.axis_index("c")
        base = wid * _BPW
        pltpu.sync_copy(idx_hbm.at[pl.ds(base, _BPW)], idx_v)
        pltpu.async_copy(table_hbm.at[idx_v], rows_v, sem).wait()
        pltpu.sync_copy(rows_v, out_hbm.at[pl.ds(base, _BPW)])

    return k(table, idx_flat)


def _mlp_body(emb_ref, w1_ref, b1_ref, w2_ref, b2_ref, w3t_hbm, b3_hbm,
              out_hbm, h2t_ref, w3t_bufs, b3_bufs, out_bufs,
              w3t_tail, b3_tail, out_tail,
              rsem, bsem, wsem, tsem):
    def start_read(slot, s):
        pltpu.make_async_copy(
            w3t_hbm.at[pl.ds(s, _PB), :], w3t_bufs.at[slot], rsem.at[slot]
        ).start()
        pltpu.make_async_copy(
            b3_hbm.at[:, pl.ds(s, _PB)], b3_bufs.at[slot], bsem.at[slot]
        ).start()

    for k in range(_RBUF):
        start_read(k, k * _PB)
    pltpu.make_async_copy(
        w3t_hbm.at[pl.ds(_TAILOFF, _TAILW), :], w3t_tail, tsem.at[0]).start()
    pltpu.make_async_copy(
        b3_hbm.at[:, pl.ds(_TAILOFF, _TAILW)], b3_tail, tsem.at[1]).start()

    h1t = jnp.maximum(
        lax.dot_general(w1_ref[...], emb_ref[...], (((0,), (1,)), ((), ())),
                        preferred_element_type=jnp.float32) + b1_ref[...],
        0.0)
    h2t = jnp.maximum(
        lax.dot_general(w2_ref[...], h1t, (((0,), (0,)), ((), ())),
                        preferred_element_type=jnp.float32) + b2_ref[...],
        0.0)
    h2t_ref[...] = h2t.astype(jnp.bfloat16)

    pltpu.make_async_copy(
        w3t_hbm.at[pl.ds(_TAILOFF, _TAILW), :], w3t_tail, tsem.at[0]).wait()
    pltpu.make_async_copy(
        b3_hbm.at[:, pl.ds(_TAILOFF, _TAILW)], b3_tail, tsem.at[1]).wait()
    out_tail[...] = (
        jnp.dot(w3t_tail[...].astype(jnp.bfloat16), h2t_ref[...],
                preferred_element_type=jnp.float32)
        + jnp.transpose(b3_tail[...], (1, 0)))
    pltpu.make_async_copy(
        out_tail, out_hbm.at[pl.ds(_TAILOFF, _TAILW), :], tsem.at[2]).start()

    def wait_read(slot):
        pltpu.make_async_copy(
            w3t_hbm.at[pl.ds(0, _PB), :], w3t_bufs.at[slot], rsem.at[slot]
        ).wait()
        pltpu.make_async_copy(
            b3_hbm.at[:, pl.ds(0, _PB)], b3_bufs.at[slot], bsem.at[slot]
        ).wait()

    def wait_write(slot):
        pltpu.make_async_copy(
            out_bufs.at[slot], out_hbm.at[pl.ds(0, _PB), :], wsem.at[slot]
        ).wait()

    def compute_and_write(rslot, wslot, s):
        out_bufs[wslot] = (
            jnp.dot(w3t_bufs[rslot].astype(jnp.bfloat16), h2t_ref[...],
                    preferred_element_type=jnp.float32)
            + jnp.transpose(b3_bufs[rslot], (1, 0)))
        pltpu.make_async_copy(
            out_bufs.at[wslot], out_hbm.at[pl.ds(s, _PB), :], wsem.at[wslot]
        ).start()

    @pl.loop(0, _NOUTER)
    def _(o):
        base = o * _RBUF
        for k in range(_RBUF):
            i = base + k
            wait_read(k)
            if k >= _WBUF:
                wait_write(k % _WBUF)
            else:
                @pl.when(o > 0)
                def _():
                    wait_write(k % _WBUF)
            compute_and_write(k, k % _WBUF, i * _PB)
            nxt = i + _RBUF
            @pl.when(nxt < _NFULL)
            def _():
                start_read(k, nxt * _PB)

    for e in range(_NEPI):
        wait_read(e)
        wait_write(e % _WBUF)
        compute_and_write(e, e % _WBUF, (_MAIN + e) * _PB)

    for k in range(_WBUF):
        wait_write(k)
    pltpu.make_async_copy(
        out_tail, out_hbm.at[pl.ds(_TAILOFF, _TAILW), :], tsem.at[2]).wait()


def _mlp_tc(emb, W1, b1, W2, b2, W3T, b3):
    out_t = pl.pallas_call(
        _mlp_body,
        in_specs=[
            pl.BlockSpec((B, 2 * H), lambda: (0, 0)),
            pl.BlockSpec((2 * H, H), lambda: (0, 0)),
            pl.BlockSpec((H, 1), lambda: (0, 0)),
            pl.BlockSpec((H, H), lambda: (0, 0)),
            pl.BlockSpec((H, 1), lambda: (0, 0)),
            pl.BlockSpec(memory_space=pl.ANY),
            pl.BlockSpec(memory_space=pl.ANY),
        ],
        out_specs=pl.BlockSpec(memory_space=pl.ANY),
        out_shape=jax.ShapeDtypeStruct((P, B), jnp.float32),
        scratch_shapes=[
            pltpu.VMEM((H, B), jnp.bfloat16),
            pltpu.VMEM((_RBUF, _PB, H), jnp.float32),
            pltpu.VMEM((_RBUF, 1, _PB), jnp.float32),
            pltpu.VMEM((_WBUF, _PB, B), jnp.float32),
            pltpu.VMEM((_TAILW, H), jnp.float32),
            pltpu.VMEM((1, _TAILW), jnp.float32),
            pltpu.VMEM((_TAILW, B), jnp.float32),
            pltpu.SemaphoreType.DMA((_RBUF,)),
            pltpu.SemaphoreType.DMA((_RBUF,)),
            pltpu.SemaphoreType.DMA((_WBUF,)),
            pltpu.SemaphoreType.DMA((3,)),
        ],
    )(emb, W1, b1.reshape(H, 1), W2, b2.reshape(H, 1), W3T,
      b3.reshape(1, P))
    return out_t.T


def kernel(x, table, W1, b1, W2, b2, W3, b3):
    idx_flat = x.reshape(-1).astype(jnp.int32)
    emb = _sc_gather(table, idx_flat).reshape(B, 2 * H)
    return _mlp_tc(emb, W1, b1, W2, b2, W3.T, b3)


# speedup vs baseline: 1.0085x; 1.0040x over previous
import functools

import jax
import jax.numpy as jnp
from jax import lax
from jax.experimental import pallas as pl
from jax.experimental.pallas import tpu as pltpu
from jax.experimental.pallas import tpu_sc as plsc

P = 100000
H = 128
B = 1024

_NC = 2
_NS = 16
_NW = _NC * _NS
_NIDX = 2 * B
_BPW = _NIDX // _NW

_PB = 3072
_NFULL = P // _PB
_RBUF = 3
_WBUF = 3
_MAIN = 30
_NEPI = _NFULL - _MAIN
_NOUTER = _MAIN // _RBUF
_TAILW = P - _NFULL * _PB
_TAILOFF = _NFULL * _PB


def _sc_gather(table, idx_flat):
    mesh = plsc.VectorSubcoreMesh(core_axis_name="c", subcore_axis_name="s")

    @functools.partial(
        pl.kernel,
        mesh=mesh,
        out_type=jax.ShapeDtypeStruct((_NIDX, H), jnp.float32),
        scratch_types=[
            pltpu.VMEM((_BPW,), jnp.int32),
            pltpu.VMEM((_BPW, H), jnp.float32),
            pltpu.SemaphoreType.DMA,
        ],
    )
    def k(table_hbm, idx_hbm, out_hbm, idx_v, rows_v, sem):
        wid = lax.axis_index("s") * _NC + lax.axis_index("c")
        base = wid * _BPW
        pltpu.sync_copy(idx_hbm.at[pl.ds(base, _BPW)], idx_v)
        pltpu.async_copy(table_hbm.at[idx_v], rows_v, sem).wait()
        pltpu.sync_copy(rows_v, out_hbm.at[pl.ds(base, _BPW)])

    return k(table, idx_flat)


def _mlp_body(emb_ref, w1_ref, b1_ref, w2_ref, b2_ref, w3t_hbm, b3_hbm,
              out_hbm, h2t_ref, w3t_bufs, b3_bufs, out_bufs,
              w3t_tail, b3_tail, out_tail,
              rsem, bsem, wsem, tsem):
    def start_read(slot, s):
        pltpu.make_async_copy(
            w3t_hbm.at[pl.ds(s, _PB), :], w3t_bufs.at[slot], rsem.at[slot]
        ).start()
        pltpu.make_async_copy(
            b3_hbm.at[:, pl.ds(s, _PB)], b3_bufs.at[slot], bsem.at[slot]
        ).start()

    for k in range(_RBUF):
        start_read(k, k * _PB)
    pltpu.make_async_copy(
        w3t_hbm.at[pl.ds(_TAILOFF, _TAILW), :], w3t_tail, tsem.at[0]).start()
    pltpu.make_async_copy(
        b3_hbm.at[:, pl.ds(_TAILOFF, _TAILW)], b3_tail, tsem.at[1]).start()

    h1t = jnp.maximum(
        lax.dot_general(w1_ref[...], emb_ref[...], (((0,), (1,)), ((), ())),
                        preferred_element_type=jnp.float32) + b1_ref[...],
        0.0)
    h2t = jnp.maximum(
        lax.dot_general(w2_ref[...], h1t, (((0,), (0,)), ((), ())),
                        preferred_element_type=jnp.float32) + b2_ref[...],
        0.0)
    h2t_ref[...] = h2t.astype(jnp.bfloat16)

    def wait_read(slot):
        pltpu.make_async_copy(
            w3t_hbm.at[pl.ds(0, _PB), :], w3t_bufs.at[slot], rsem.at[slot]
        ).wait()
        pltpu.make_async_copy(
            b3_hbm.at[:, pl.ds(0, _PB)], b3_bufs.at[slot], bsem.at[slot]
        ).wait()

    def wait_write(slot):
        pltpu.make_async_copy(
            out_bufs.at[slot], out_hbm.at[pl.ds(0, _PB), :], wsem.at[slot]
        ).wait()

    def compute_and_write(rslot, wslot, s):
        out_bufs[wslot] = (
            jnp.dot(w3t_bufs[rslot].astype(jnp.bfloat16), h2t_ref[...],
                    preferred_element_type=jnp.float32)
            + jnp.transpose(b3_bufs[rslot], (1, 0)))
        pltpu.make_async_copy(
            out_bufs.at[wslot], out_hbm.at[pl.ds(s, _PB), :], wsem.at[wslot]
        ).start()

    @pl.loop(0, _NOUTER)
    def _(o):
        base = o * _RBUF
        for k in range(_RBUF):
            i = base + k
            wait_read(k)
            if k >= _WBUF:
                wait_write(k % _WBUF)
            else:
                @pl.when(o > 0)
                def _():
                    wait_write(k % _WBUF)
            compute_and_write(k, k % _WBUF, i * _PB)
            nxt = i + _RBUF
            @pl.when(nxt < _NFULL)
            def _():
                start_read(k, nxt * _PB)

    for e in range(_NEPI):
        wait_read(e)
        wait_write(e % _WBUF)
        compute_and_write(e, e % _WBUF, (_MAIN + e) * _PB)

    pltpu.make_async_copy(
        w3t_hbm.at[pl.ds(_TAILOFF, _TAILW), :], w3t_tail, tsem.at[0]).wait()
    pltpu.make_async_copy(
        b3_hbm.at[:, pl.ds(_TAILOFF, _TAILW)], b3_tail, tsem.at[1]).wait()
    out_tail[...] = (
        jnp.dot(w3t_tail[...].astype(jnp.bfloat16), h2t_ref[...],
                preferred_element_type=jnp.float32)
        + jnp.transpose(b3_tail[...], (1, 0)))
    pltpu.make_async_copy(
        out_tail, out_hbm.at[pl.ds(_TAILOFF, _TAILW), :], tsem.at[2]).start()

    for k in range(_WBUF):
        wait_write(k)
    pltpu.make_async_copy(
        out_tail, out_hbm.at[pl.ds(_TAILOFF, _TAILW), :], tsem.at[2]).wait()


def _mlp_tc(emb, W1, b1, W2, b2, W3T, b3):
    out_t = pl.pallas_call(
        _mlp_body,
        in_specs=[
            pl.BlockSpec((B, 2 * H), lambda: (0, 0)),
            pl.BlockSpec((2 * H, H), lambda: (0, 0)),
            pl.BlockSpec((H, 1), lambda: (0, 0)),
            pl.BlockSpec((H, H), lambda: (0, 0)),
            pl.BlockSpec((H, 1), lambda: (0, 0)),
            pl.BlockSpec(memory_space=pl.ANY),
            pl.BlockSpec(memory_space=pl.ANY),
        ],
        out_specs=pl.BlockSpec(memory_space=pl.ANY),
        out_shape=jax.ShapeDtypeStruct((P, B), jnp.float32),
        scratch_shapes=[
            pltpu.VMEM((H, B), jnp.bfloat16),
            pltpu.VMEM((_RBUF, _PB, H), jnp.float32),
            pltpu.VMEM((_RBUF, 1, _PB), jnp.float32),
            pltpu.VMEM((_WBUF, _PB, B), jnp.float32),
            pltpu.VMEM((_TAILW, H), jnp.float32),
            pltpu.VMEM((1, _TAILW), jnp.float32),
            pltpu.VMEM((_TAILW, B), jnp.float32),
            pltpu.SemaphoreType.DMA((_RBUF,)),
            pltpu.SemaphoreType.DMA((_RBUF,)),
            pltpu.SemaphoreType.DMA((_WBUF,)),
            pltpu.SemaphoreType.DMA((3,)),
        ],
    )(emb, W1, b1.reshape(H, 1), W2, b2.reshape(H, 1), W3T,
      b3.reshape(1, P))
    return out_t.T


def kernel(x, table, W1, b1, W2, b2, W3, b3):
    idx_flat = x.reshape(-1).astype(jnp.int32)
    emb = _sc_gather(table, idx_flat).reshape(B, 2 * H)
    return _mlp_tc(emb, W1, b1, W2, b2, W3.T, b3)
